# baseline (speedup 1.0000x reference)
import dataclasses
import functools

import jax
import jax.numpy as jnp
from jax import lax
from jax.experimental import pallas as pl
from jax.experimental.pallas import tpu as pltpu
from jax.experimental.pallas import tpu_sc as plsc

NUM_OUTPUTS = 128
LANES = 16
NVREG = NUM_OUTPUTS // LANES
NUM_CORES = 2
NUM_SUBCORES = 16
NW = NUM_CORES * NUM_SUBCORES

CHUNK = 16
GATHER_SPLIT = 4


def _make_kernel(batch, max_active):
    rows_per_w = batch // NW
    nchunk = rows_per_w // CHUNK
    nfeat = batch * max_active
    feats_per_w = rows_per_w * max_active
    feats_per_chunk = CHUNK * max_active
    gwin = feats_per_chunk // GATHER_SPLIT
    assert feats_per_chunk % GATHER_SPLIT == 0
    assert gwin % 8 == 0 and gwin <= 128
    assert batch % (NW * CHUNK) == 0
    assert nchunk % 2 == 0
    assert max_active <= 2 * LANES

    mesh = plsc.VectorSubcoreMesh(core_axis_name="c", subcore_axis_name="s")
    out_sds = jax.ShapeDtypeStruct((batch, NUM_OUTPUTS), jnp.float32)
    idx_buf = pltpu.VMEM((feats_per_w,), jnp.int32)
    vals_buf = pltpu.VMEM((feats_per_chunk,), jnp.int32)
    row_buf = pltpu.VMEM((feats_per_chunk, NUM_OUTPUTS), jnp.float32)
    out_buf = pltpu.VMEM((CHUNK, NUM_OUTPUTS), jnp.float32)

    cp = pltpu.CompilerParams()
    if "needs_layout_passes" in pltpu.CompilerParams.__dataclass_fields__:
        cp = dataclasses.replace(cp, needs_layout_passes=False)

    @functools.partial(
        pl.kernel,
        out_type=(out_sds, out_sds),
        mesh=mesh,
        compiler_params=cp,
        scratch_types=[
            idx_buf, idx_buf,
            vals_buf, vals_buf,
            row_buf, row_buf,
            out_buf, out_buf,
            pltpu.VMEM((NUM_OUTPUTS,), jnp.float32),
            pltpu.SemaphoreType.DMA,
            pltpu.SemaphoreType.DMA,
            pltpu.SemaphoreType.DMA,
            pltpu.SemaphoreType.DMA,
        ],
    )
    def k(packed_hbm, w_hbm, bias_hbm,
          out0_hbm, out1_hbm,
          idx0_v, idx1_v, vals_a, vals_b, rows_a, rows_b, out_a, out_b,
          bias_v, sem_ga, sem_gb, sem_oa, sem_ob):
        wid = lax.axis_index("s") * NUM_CORES + lax.axis_index("c")
        base_feat = wid * feats_per_w
        idx_stage0 = pltpu.async_copy(
            packed_hbm.at[pl.ds(base_feat, feats_per_w)], idx0_v, sem_ga)
        idx_stage1 = pltpu.async_copy(
            packed_hbm.at[pl.ds(2 * nfeat + base_feat, feats_per_w)],
            idx1_v, sem_gb)
        pltpu.sync_copy(bias_hbm, bias_v)
        idx_stage0.wait()
        idx_stage1.wait()

        def fire(idx_v, vals_off, c, vals_v, rows_v, sem):
            pltpu.async_copy(
                packed_hbm.at[pl.ds(vals_off + c * feats_per_chunk,
                                    feats_per_chunk)],
                vals_v, sem)
            for g in range(GATHER_SPLIT):
                pltpu.async_copy(
                    w_hbm.at[idx_v.at[pl.ds(c * feats_per_chunk + g * gwin,
                                            gwin)]],
                    rows_v.at[pl.ds(g * gwin, gwin)],
                    sem,
                )

        def drain_in(vals_v, rows_v, sem):
            pltpu.make_async_copy(
                w_hbm.at[pl.ds(0, feats_per_chunk)], rows_v, sem).wait()
            pltpu.make_async_copy(
                packed_hbm.at[pl.ds(0, feats_per_chunk)], vals_v, sem).wait()

        def drain_out(out_hbm, out_v, sem):
            pltpu.make_async_copy(out_hbm.at[pl.ds(0, CHUNK)], out_v, sem).wait()

        def compute(vals_v, rows_v, out_v, out_hbm, c, sem):
            bias_r = [bias_v[pl.ds(kk * LANES, LANES)] for kk in range(NVREG)]

            @pl.loop(0, CHUNK)
            def _(r):
                acc = list(bias_r)
                rbase = r * max_active
                v0 = plsc.bitcast(vals_v[pl.ds(rbase, LANES)], jnp.float32)
                v1 = plsc.bitcast(
                    vals_v[pl.ds(rbase + max_active - LANES, LANES)],
                    jnp.float32)
                for j in range(max_active):
                    s = (v0[j] if j < LANES
                         else v1[j - (max_active - LANES)])
                    v = jnp.broadcast_to(s, (LANES,))
                    for kk in range(NVREG):
                        acc[kk] = acc[kk] + v * rows_v[rbase + j,
                                                       pl.ds(kk * LANES, LANES)]
                for kk in range(NVREG):
                    out_v[r, pl.ds(kk * LANES, LANES)] = acc[kk]

            pltpu.async_copy(
                out_v,
                out_hbm.at[pl.ds(wid * rows_per_w + c * CHUNK, CHUNK)],
                sem)

        for idx_v, vals_off, out_hbm in (
            (idx0_v, nfeat + base_feat, out0_hbm),
            (idx1_v, 3 * nfeat + base_feat, out1_hbm),
        ):
            fire(idx_v, vals_off, 0, vals_a, rows_a, sem_ga)

            @pl.loop(0, nchunk, step=2)
            def _(c):
                fire(idx_v, vals_off, c + 1, vals_b, rows_b, sem_gb)
                drain_in(vals_a, rows_a, sem_ga)

                @pl.when(c > 0)
                def _():
                    drain_out(out_hbm, out_a, sem_oa)
                compute(vals_a, rows_a, out_a, out_hbm, c, sem_oa)

                @pl.when(c + 2 < nchunk)
                def _():
                    fire(idx_v, vals_off, c + 2, vals_a, rows_a, sem_ga)
                drain_in(vals_b, rows_b, sem_gb)

                @pl.when(c > 0)
                def _():
                    drain_out(out_hbm, out_b, sem_ob)
                compute(vals_b, rows_b, out_b, out_hbm, c + 1, sem_ob)

            drain_out(out_hbm, out_a, sem_oa)
            drain_out(out_hbm, out_b, sem_ob)

    return k


def kernel(feature_indices_0, feature_values_0, feature_indices_1,
           feature_values_1, weight, bias):
    batch, max_active = feature_indices_0.shape
    packed = jnp.concatenate([
        feature_indices_0.reshape(-1),
        jax.lax.bitcast_convert_type(feature_values_0, jnp.int32).reshape(-1),
        feature_indices_1.reshape(-1),
        jax.lax.bitcast_convert_type(feature_values_1, jnp.int32).reshape(-1),
    ])
    k = _make_kernel(batch, max_active)
    out0, out1 = k(packed, weight, bias)
    return (out0, out1)


# speedup vs baseline: 1.0898x; 1.0898x over previous
import dataclasses
import functools

import jax
import jax.numpy as jnp
from jax import lax
from jax.experimental import pallas as pl
from jax.experimental.pallas import tpu as pltpu
from jax.experimental.pallas import tpu_sc as plsc

NUM_OUTPUTS = 128
LANES = 16
NVREG = NUM_OUTPUTS // LANES
NUM_CORES = 2
NUM_SUBCORES = 16
NW = NUM_CORES * NUM_SUBCORES

CHUNK = 16
PADL = 128


def _make_kernel(batch, max_active):
    rows_per_w = batch // NW
    nchunk = rows_per_w // CHUNK
    rows_per_chunk = CHUNK * max_active
    assert batch % (NW * CHUNK) == 0
    assert nchunk % 2 == 0
    assert max_active <= PADL

    mesh = plsc.VectorSubcoreMesh(core_axis_name="c", subcore_axis_name="s")
    out_sds = jax.ShapeDtypeStruct((batch, NUM_OUTPUTS), jnp.float32)
    idx_buf = pltpu.VMEM((CHUNK, PADL), jnp.int32)
    vals_buf = pltpu.VMEM((CHUNK, PADL), jnp.float32)
    row_buf = pltpu.VMEM((rows_per_chunk, NUM_OUTPUTS), jnp.float32)
    out_buf = pltpu.VMEM((CHUNK, NUM_OUTPUTS), jnp.float32)

    cp = pltpu.CompilerParams()
    if "needs_layout_passes" in pltpu.CompilerParams.__dataclass_fields__:
        cp = dataclasses.replace(cp, needs_layout_passes=False)

    @functools.partial(
        pl.kernel,
        out_type=(out_sds, out_sds),
        mesh=mesh,
        compiler_params=cp,
        scratch_types=[
            idx_buf, idx_buf,
            vals_buf, vals_buf,
            row_buf, row_buf,
            out_buf, out_buf,
            pltpu.VMEM((NUM_OUTPUTS,), jnp.float32),
            pltpu.SemaphoreType.DMA,
            pltpu.SemaphoreType.DMA,
            pltpu.SemaphoreType.DMA,
            pltpu.SemaphoreType.DMA,
            pltpu.SemaphoreType.DMA,
            pltpu.SemaphoreType.DMA,
            pltpu.SemaphoreType.DMA,
            pltpu.SemaphoreType.DMA,
        ],
    )
    def k(idx0_hbm, vals0_hbm, idx1_hbm, vals1_hbm, w_hbm, bias_hbm,
          out0_hbm, out1_hbm,
          idx_a, idx_b, vals_a, vals_b, rows_a, rows_b, out_a, out_b,
          bias_v, sem_ia, sem_ib, sem_va, sem_vb,
          sem_ga, sem_gb, sem_oa, sem_ob):
        wid = lax.axis_index("s") * NUM_CORES + lax.axis_index("c")
        base_row = wid * rows_per_w
        pltpu.sync_copy(bias_hbm, bias_v)

        def fire_idx(idx_hbm, c, idx_v, sem):
            pltpu.async_copy(
                idx_hbm.at[pl.ds(base_row + c * CHUNK, CHUNK)], idx_v, sem)

        def drain_idx(idx_hbm, idx_v, sem):
            pltpu.make_async_copy(
                idx_hbm.at[pl.ds(0, CHUNK)], idx_v, sem).wait()

        def fire_vals(vals_hbm, c, vals_v, sem):
            pltpu.async_copy(
                vals_hbm.at[pl.ds(base_row + c * CHUNK, CHUNK)], vals_v, sem)

        def drain_vals(vals_hbm, vals_v, sem):
            pltpu.make_async_copy(
                vals_hbm.at[pl.ds(0, CHUNK)], vals_v, sem).wait()

        def fire_gather(idx_v, rows_v, sem):
            for rr in range(CHUNK):
                pltpu.async_copy(
                    w_hbm.at[idx_v.at[rr, pl.ds(0, max_active)]],
                    rows_v.at[pl.ds(rr * max_active, max_active)],
                    sem,
                )

        def drain_rows(rows_v, sem):
            pltpu.make_async_copy(
                w_hbm.at[pl.ds(0, rows_per_chunk)], rows_v, sem).wait()

        def drain_out(out_hbm, out_v, sem):
            pltpu.make_async_copy(out_hbm.at[pl.ds(0, CHUNK)], out_v, sem).wait()

        def compute(vals_v, rows_v, out_v, out_hbm, c, sem):
            bias_r = [bias_v[pl.ds(kk * LANES, LANES)] for kk in range(NVREG)]

            @pl.loop(0, CHUNK)
            def _(r):
                acc = list(bias_r)
                rbase = r * max_active
                v0 = vals_v[r, pl.ds(0, LANES)]
                v1 = vals_v[r, pl.ds(LANES, LANES)]
                for j in range(max_active):
                    s = v0[j] if j < LANES else v1[j - LANES]
                    v = jnp.broadcast_to(s, (LANES,))
                    for kk in range(NVREG):
                        acc[kk] = acc[kk] + v * rows_v[rbase + j,
                                                       pl.ds(kk * LANES, LANES)]
                for kk in range(NVREG):
                    out_v[r, pl.ds(kk * LANES, LANES)] = acc[kk]

            pltpu.async_copy(
                out_v,
                out_hbm.at[pl.ds(base_row + c * CHUNK, CHUNK)],
                sem)

        for idx_hbm, vals_hbm, out_hbm in (
            (idx0_hbm, vals0_hbm, out0_hbm),
            (idx1_hbm, vals1_hbm, out1_hbm),
        ):
            fire_idx(idx_hbm, 0, idx_a, sem_ia)
            fire_vals(vals_hbm, 0, vals_a, sem_va)
            fire_vals(vals_hbm, 1, vals_b, sem_vb)
            drain_idx(idx_hbm, idx_a, sem_ia)
            fire_gather(idx_a, rows_a, sem_ga)
            fire_idx(idx_hbm, 1, idx_b, sem_ib)

            @pl.loop(0, nchunk, step=2)
            def _(c):
                drain_idx(idx_hbm, idx_b, sem_ib)
                fire_gather(idx_b, rows_b, sem_gb)

                drain_rows(rows_a, sem_ga)

                @pl.when(c + 2 < nchunk)
                def _():
                    fire_idx(idx_hbm, c + 2, idx_a, sem_ia)

                @pl.when(c > 0)
                def _():
                    drain_out(out_hbm, out_a, sem_oa)
                drain_vals(vals_hbm, vals_a, sem_va)
                compute(vals_a, rows_a, out_a, out_hbm, c, sem_oa)

                @pl.when(c + 2 < nchunk)
                def _():
                    fire_vals(vals_hbm, c + 2, vals_a, sem_va)
                    drain_idx(idx_hbm, idx_a, sem_ia)
                    fire_gather(idx_a, rows_a, sem_ga)

                drain_rows(rows_b, sem_gb)

                @pl.when(c + 3 < nchunk)
                def _():
                    fire_idx(idx_hbm, c + 3, idx_b, sem_ib)

                @pl.when(c > 0)
                def _():
                    drain_out(out_hbm, out_b, sem_ob)
                drain_vals(vals_hbm, vals_b, sem_vb)
                compute(vals_b, rows_b, out_b, out_hbm, c + 1, sem_ob)

                @pl.when(c + 3 < nchunk)
                def _():
                    fire_vals(vals_hbm, c + 3, vals_b, sem_vb)

            drain_out(out_hbm, out_a, sem_oa)
            drain_out(out_hbm, out_b, sem_ob)

    return k


def kernel(feature_indices_0, feature_values_0, feature_indices_1,
           feature_values_1, weight, bias):
    batch, max_active = feature_indices_0.shape
    padw = ((0, 0), (0, PADL - max_active))
    k = _make_kernel(batch, max_active)
    out0, out1 = k(
        jnp.pad(feature_indices_0, padw), jnp.pad(feature_values_0, padw),
        jnp.pad(feature_indices_1, padw), jnp.pad(feature_values_1, padw),
        weight, bias,
    )
    return (out0, out1)


# speedup vs baseline: 1.0956x; 1.0053x over previous
import dataclasses
import functools

import jax
import jax.numpy as jnp
from jax import lax
from jax.experimental import pallas as pl
from jax.experimental.pallas import tpu as pltpu
from jax.experimental.pallas import tpu_sc as plsc

NUM_OUTPUTS = 128
LANES = 16
NVREG = NUM_OUTPUTS // LANES
NUM_CORES = 2
NUM_SUBCORES = 16
NW = NUM_CORES * NUM_SUBCORES

CHUNK = 16
PADL = 128


def _make_kernel(batch, max_active):
    rows_per_w = batch // NW
    nchunk = rows_per_w // CHUNK
    rows_per_chunk = CHUNK * max_active
    assert batch % (NW * CHUNK) == 0
    assert nchunk % 2 == 0
    assert max_active <= PADL

    mesh = plsc.VectorSubcoreMesh(core_axis_name="c", subcore_axis_name="s")
    out_sds = jax.ShapeDtypeStruct((batch, NUM_OUTPUTS), jnp.float32)
    idx_buf = pltpu.VMEM((CHUNK, PADL), jnp.int32)
    vals_buf = pltpu.VMEM((CHUNK, PADL), jnp.float32)
    row_buf = pltpu.VMEM((rows_per_chunk, NUM_OUTPUTS), jnp.float32)
    out_buf = pltpu.VMEM((CHUNK, NUM_OUTPUTS), jnp.float32)

    cp = pltpu.CompilerParams()
    if "needs_layout_passes" in pltpu.CompilerParams.__dataclass_fields__:
        cp = dataclasses.replace(cp, needs_layout_passes=False)

    @functools.partial(
        pl.kernel,
        out_type=(out_sds, out_sds),
        mesh=mesh,
        compiler_params=cp,
        scratch_types=[
            idx_buf, idx_buf,
            vals_buf, vals_buf,
            row_buf, row_buf,
            out_buf, out_buf,
            pltpu.VMEM((NUM_OUTPUTS,), jnp.float32),
            pltpu.SemaphoreType.DMA,
            pltpu.SemaphoreType.DMA,
            pltpu.SemaphoreType.DMA,
            pltpu.SemaphoreType.DMA,
            pltpu.SemaphoreType.DMA,
            pltpu.SemaphoreType.DMA,
            pltpu.SemaphoreType.DMA,
            pltpu.SemaphoreType.DMA,
        ],
    )
    def k(idx_hbm, vals_hbm, w_hbm, bias_hbm,
          out0_hbm, out1_hbm,
          idx_a, idx_b, vals_a, vals_b, rows_a, rows_b, out_a, out_b,
          bias_v, sem_ia, sem_ib, sem_va, sem_vb,
          sem_ga, sem_gb, sem_oa, sem_ob):
        wid = lax.axis_index("s") * NUM_CORES + lax.axis_index("c")
        base_row = wid * rows_per_w
        pltpu.sync_copy(bias_hbm, bias_v)

        def fire_idx(s_off, c, idx_v, sem):
            pltpu.async_copy(
                idx_hbm.at[pl.ds(s_off + c * CHUNK, CHUNK)], idx_v, sem)

        def drain_idx(idx_v, sem):
            pltpu.make_async_copy(
                idx_hbm.at[pl.ds(0, CHUNK)], idx_v, sem).wait()

        def fire_vals(s_off, c, vals_v, sem):
            pltpu.async_copy(
                vals_hbm.at[pl.ds(s_off + c * CHUNK, CHUNK)], vals_v, sem)

        def drain_vals(vals_v, sem):
            pltpu.make_async_copy(
                vals_hbm.at[pl.ds(0, CHUNK)], vals_v, sem).wait()

        def fire_gather(idx_v, rows_v, sem):
            for rr in range(CHUNK):
                pltpu.async_copy(
                    w_hbm.at[idx_v.at[rr, pl.ds(0, max_active)]],
                    rows_v.at[pl.ds(rr * max_active, max_active)],
                    sem,
                )

        def drain_rows(rows_v, sem):
            pltpu.make_async_copy(
                w_hbm.at[pl.ds(0, rows_per_chunk)], rows_v, sem).wait()

        def drain_out(out_hbm, out_v, sem):
            pltpu.make_async_copy(out_hbm.at[pl.ds(0, CHUNK)], out_v, sem).wait()

        def compute(vals_v, rows_v, out_v, out_hbm, c, sem):
            bias_r = [bias_v[pl.ds(kk * LANES, LANES)] for kk in range(NVREG)]

            @pl.loop(0, CHUNK)
            def _(r):
                acc = list(bias_r)
                rbase = r * max_active
                v0 = vals_v[r, pl.ds(0, LANES)]
                v1 = vals_v[r, pl.ds(LANES, LANES)]
                for j in range(max_active):
                    s = v0[j] if j < LANES else v1[j - LANES]
                    v = jnp.broadcast_to(s, (LANES,))
                    for kk in range(NVREG):
                        acc[kk] = acc[kk] + v * rows_v[rbase + j,
                                                       pl.ds(kk * LANES, LANES)]
                for kk in range(NVREG):
                    out_v[r, pl.ds(kk * LANES, LANES)] = acc[kk]

            pltpu.async_copy(
                out_v,
                out_hbm.at[pl.ds(base_row + c * CHUNK, CHUNK)],
                sem)

        for s_off, out_hbm in (
            (base_row, out0_hbm),
            (batch + base_row, out1_hbm),
        ):
            fire_idx(s_off, 0, idx_a, sem_ia)
            fire_vals(s_off, 0, vals_a, sem_va)
            fire_vals(s_off, 1, vals_b, sem_vb)
            drain_idx(idx_a, sem_ia)
            fire_gather(idx_a, rows_a, sem_ga)
            fire_idx(s_off, 1, idx_b, sem_ib)

            @pl.loop(0, nchunk, step=2)
            def _(c):
                drain_idx(idx_b, sem_ib)
                fire_gather(idx_b, rows_b, sem_gb)

                drain_rows(rows_a, sem_ga)

                @pl.when(c + 2 < nchunk)
                def _():
                    fire_idx(s_off, c + 2, idx_a, sem_ia)

                @pl.when(c > 0)
                def _():
                    drain_out(out_hbm, out_a, sem_oa)
                drain_vals(vals_a, sem_va)
                compute(vals_a, rows_a, out_a, out_hbm, c, sem_oa)

                @pl.when(c + 2 < nchunk)
                def _():
                    fire_vals(s_off, c + 2, vals_a, sem_va)
                    drain_idx(idx_a, sem_ia)
                    fire_gather(idx_a, rows_a, sem_ga)

                drain_rows(rows_b, sem_gb)

                @pl.when(c + 3 < nchunk)
                def _():
                    fire_idx(s_off, c + 3, idx_b, sem_ib)

                @pl.when(c > 0)
                def _():
                    drain_out(out_hbm, out_b, sem_ob)
                drain_vals(vals_b, sem_vb)
                compute(vals_b, rows_b, out_b, out_hbm, c + 1, sem_ob)

                @pl.when(c + 3 < nchunk)
                def _():
                    fire_vals(s_off, c + 3, vals_b, sem_vb)

            drain_out(out_hbm, out_a, sem_oa)
            drain_out(out_hbm, out_b, sem_ob)

    return k


def kernel(feature_indices_0, feature_values_0, feature_indices_1,
           feature_values_1, weight, bias):
    batch, max_active = feature_indices_0.shape
    padw = ((0, 0), (0, PADL - max_active))
    idx = jnp.pad(
        jnp.concatenate([feature_indices_0, feature_indices_1], axis=0), padw)
    vals = jnp.pad(
        jnp.concatenate([feature_values_0, feature_values_1], axis=0), padw)
    k = _make_kernel(batch, max_active)
    out0, out1 = k(idx, vals, weight, bias)
    return (out0, out1)
